# baseline (speedup 1.0000x reference)
import jax
import jax.numpy as jnp
from jax import lax
from jax.experimental import pallas as pl
from jax.experimental.pallas import tpu as pltpu
from jax.experimental.pallas import tpu_sc as plsc

N_ENTITIES = 1000000
N_RELATIONS = 3
DIMS = 32
BATCH = 16384

NC = 2
NS = 16
NW = NC * NS
LANES = 16

B_PER_W = BATCH // NW
CHUNK = 128
N_CHUNKS = B_PER_W // CHUNK
N_BLOCKS = B_PER_W // LANES


def _body(h_hbm, r_hbm, t_hbm, ent_hbm, rel_hbm, bh_hbm, bt_hbm, out_hbm,
          h_v, r_v, t_v, lhs_v, rhs_v, bh_v, bt_v, rel_v, out_v, sem):
    wid = lax.axis_index("s") * NC + lax.axis_index("c")
    base = wid * B_PER_W

    pltpu.sync_copy(h_hbm.at[pl.ds(base, B_PER_W)], h_v)
    pltpu.sync_copy(t_hbm.at[pl.ds(base, B_PER_W)], t_v)
    pltpu.sync_copy(r_hbm.at[pl.ds(base, B_PER_W)], r_v)
    pltpu.sync_copy(rel_hbm, rel_v)

    copies = []
    for j in range(N_CHUNKS):
        s = pl.ds(j * CHUNK, CHUNK)
        copies.append(pltpu.async_copy(ent_hbm.at[h_v.at[s]], lhs_v.at[s], sem))
        copies.append(pltpu.async_copy(ent_hbm.at[t_v.at[s]], rhs_v.at[s], sem))
        copies.append(pltpu.async_copy(bh_hbm.at[h_v.at[s]], bh_v.at[s], sem))
        copies.append(pltpu.async_copy(bt_hbm.at[t_v.at[s]], bt_v.at[s], sem))
    for c in copies:
        c.wait()

    lane_iota = lax.iota(jnp.int32, LANES)

    def block(blk, carry):
        o = blk * LANES
        rows = lane_iota + o
        rvec = r_v[pl.ds(o, LANES)]
        acc = bh_v[pl.ds(o, LANES)] + bt_v[pl.ds(o, LANES)]
        for d in range(DIMS):
            dv = jnp.full((LANES,), d, jnp.int32)
            lv = plsc.load_gather(lhs_v, [rows, dv])
            rv = plsc.load_gather(rhs_v, [rows, dv])
            relv = plsc.load_gather(rel_v, [rvec, dv])
            acc = acc + (lv + relv) * rv
        out_v[pl.ds(o, LANES)] = acc
        return carry

    lax.fori_loop(0, N_BLOCKS, block, 0)
    pltpu.sync_copy(out_v, out_hbm.at[pl.ds(base, B_PER_W)])


@jax.jit
def _run(h, r, t, entities, relations, bh, bt):
    kfn = pl.kernel(
        _body,
        out_type=jax.ShapeDtypeStruct((BATCH,), jnp.float32),
        mesh=plsc.VectorSubcoreMesh(core_axis_name="c", subcore_axis_name="s"),
        compiler_params=pltpu.CompilerParams(
            needs_layout_passes=False, use_tc_tiling_on_sc=False),
        scratch_types=[
            pltpu.VMEM((B_PER_W,), jnp.int32),
            pltpu.VMEM((B_PER_W,), jnp.int32),
            pltpu.VMEM((B_PER_W,), jnp.int32),
            pltpu.VMEM((B_PER_W, DIMS), jnp.float32),
            pltpu.VMEM((B_PER_W, DIMS), jnp.float32),
            pltpu.VMEM((B_PER_W,), jnp.float32),
            pltpu.VMEM((B_PER_W,), jnp.float32),
            pltpu.VMEM((N_RELATIONS, DIMS), jnp.float32),
            pltpu.VMEM((B_PER_W,), jnp.float32),
            pltpu.SemaphoreType.DMA,
        ],
    )
    return kfn(h, r, t, entities, relations, bh, bt)


def kernel(input_tensor, entities, relations, bias_head, bias_tail):
    h = input_tensor[:, 0].astype(jnp.int32)
    r = input_tensor[:, 1].astype(jnp.int32)
    t = input_tensor[:, 2].astype(jnp.int32)
    out = _run(h, r, t, entities, relations,
               bias_head.reshape(-1), bias_tail.reshape(-1))
    return out.reshape(BATCH, 1)
